# baseline (speedup 1.0000x reference)
import jax
import jax.numpy as jnp
from jax.experimental import pallas as pl
from jax.experimental.pallas import tpu as pltpu

_C, _H, _W = 256, 64, 64
_HALF = _C // 2


def _make_body(b):
    def _body(rw_ref, cw_ref, out_ref, pos_ref, sem):
        cw_t = cw_ref[:_W, :].T
        rw_t = rw_ref[:_H, :].T
        pos_ref[:_HALF] = jnp.broadcast_to(cw_t[:, None, :], (_HALF, _H, _W))
        pos_ref[_HALF:] = jnp.broadcast_to(rw_t[:, :, None], (_HALF, _H, _W))
        copies = [
            pltpu.make_async_copy(pos_ref, out_ref.at[i], sem.at[i])
            for i in range(b)
        ]
        for cp in copies:
            cp.start()
        for cp in copies:
            cp.wait()
    return _body


def kernel(x, row_weight, col_weight):
    b = x.shape[0]
    return pl.pallas_call(
        _make_body(b),
        in_specs=[
            pl.BlockSpec(memory_space=pltpu.VMEM),
            pl.BlockSpec(memory_space=pltpu.VMEM),
        ],
        out_specs=pl.BlockSpec(memory_space=pl.ANY),
        out_shape=jax.ShapeDtypeStruct((b, _C, _H, _W), jnp.float32),
        scratch_shapes=[
            pltpu.VMEM((_C, _H, _W), jnp.float32),
            pltpu.SemaphoreType.DMA((16,)),
        ],
    )(row_weight, col_weight)


# speedup vs baseline: 1.7592x; 1.7592x over previous
import jax
import jax.numpy as jnp
from jax.experimental import pallas as pl
from jax.experimental.pallas import tpu as pltpu

_C, _H, _W = 256, 64, 64
_HALF = _C // 2
_HW = _H * _W


def _make_body(b):
    def _body(rw_ref, cw_ref, out_ref, pos_ref, sem):
        k = jax.lax.broadcasted_iota(jnp.int32, (_HALF, _HW), 0)
        j = jax.lax.broadcasted_iota(jnp.int32, (_HALF, _HW), 1)
        sel_top = (j % _W == k).astype(jnp.float32)
        sel_bot = (j // _W == k).astype(jnp.float32)
        dims = (((0,), (0,)), ((), ()))
        pos_ref[:_HALF, :] = jax.lax.dot_general(
            cw_ref[...], sel_top, dims, preferred_element_type=jnp.float32)
        pos_ref[_HALF:, :] = jax.lax.dot_general(
            rw_ref[...], sel_bot, dims, preferred_element_type=jnp.float32)
        copies = [
            pltpu.make_async_copy(pos_ref, out_ref.at[i], sem.at[i])
            for i in range(b)
        ]
        for cp in copies:
            cp.start()
        for cp in copies:
            cp.wait()
    return _body


def kernel(x, row_weight, col_weight):
    b = x.shape[0]
    out = pl.pallas_call(
        _make_body(b),
        in_specs=[
            pl.BlockSpec(memory_space=pltpu.VMEM),
            pl.BlockSpec(memory_space=pltpu.VMEM),
        ],
        out_specs=pl.BlockSpec(memory_space=pl.ANY),
        out_shape=jax.ShapeDtypeStruct((b, _C, _HW), jnp.float32),
        scratch_shapes=[
            pltpu.VMEM((_C, _HW), jnp.float32),
            pltpu.SemaphoreType.DMA((b,)),
        ],
    )(row_weight, col_weight)
    return out.reshape(b, _C, _H, _W)


# speedup vs baseline: 5.7508x; 3.2690x over previous
import jax
import jax.numpy as jnp
from jax.experimental import pallas as pl
from jax.experimental.pallas import tpu as pltpu

_C, _H, _W = 256, 64, 64
_HALF = _C // 2
_HW = _H * _W


def _make_body(b):
    def _body(rw_ref, cw_ref, out_ref, pos_ref, sem):
        j = jax.lax.broadcasted_iota(jnp.int32, (_HW, _W), 0)
        k = jax.lax.broadcasted_iota(jnp.int32, (_HW, _W), 1)
        sel_w = (j % _W == k).astype(jnp.float32)
        sel_h = (j // _W == k).astype(jnp.float32)
        left = jax.lax.dot(
            sel_w, cw_ref[:_W, :], precision=jax.lax.Precision.HIGHEST)
        right = jax.lax.dot(
            sel_h, rw_ref[:_H, :], precision=jax.lax.Precision.HIGHEST)
        pos_ref[:, :_HALF] = left
        pos_ref[:, _HALF:] = right
        copies = [
            pltpu.make_async_copy(pos_ref, out_ref.at[i], sem.at[i])
            for i in range(b)
        ]
        for cp in copies:
            cp.start()
        for cp in copies:
            cp.wait()
    return _body


def kernel(x, row_weight, col_weight):
    b = x.shape[0]
    out = pl.pallas_call(
        _make_body(b),
        in_specs=[
            pl.BlockSpec(memory_space=pltpu.VMEM),
            pl.BlockSpec(memory_space=pltpu.VMEM),
        ],
        out_specs=pl.BlockSpec(memory_space=pl.ANY),
        out_shape=jax.ShapeDtypeStruct((b, _HW, _C), jnp.float32),
        scratch_shapes=[
            pltpu.VMEM((_HW, _C), jnp.float32),
            pltpu.SemaphoreType.DMA((b,)),
        ],
    )(row_weight, col_weight)
    return jnp.transpose(out.reshape(b, _H, _W, _C), (0, 3, 1, 2))


# speedup vs baseline: 6.0664x; 1.0549x over previous
import jax
import jax.numpy as jnp
from jax.experimental import pallas as pl
from jax.experimental.pallas import tpu as pltpu

_C, _H, _W = 256, 64, 64
_HALF = _C // 2
_HW = _H * _W


_NCHUNK = 4
_ROWS = _HW // _NCHUNK


def _make_body(b):
    def _body(rw_ref, cw_ref, out_ref, pos_ref, sem):
        copies = []
        for c in range(_NCHUNK):
            j = c * _ROWS + jax.lax.broadcasted_iota(jnp.int32, (_ROWS, _W), 0)
            k = jax.lax.broadcasted_iota(jnp.int32, (_ROWS, _W), 1)
            sel_w = (j % _W == k).astype(jnp.float32)
            sel_h = (j // _W == k).astype(jnp.float32)
            left = jax.lax.dot(
                sel_w, cw_ref[:_W, :], precision=jax.lax.Precision.HIGHEST)
            right = jax.lax.dot(
                sel_h, rw_ref[:_H, :], precision=jax.lax.Precision.HIGHEST)
            rows = pl.ds(c * _ROWS, _ROWS)
            pos_ref[rows, :_HALF] = left
            pos_ref[rows, _HALF:] = right
            for i in range(b):
                cp = pltpu.make_async_copy(
                    pos_ref.at[rows], out_ref.at[i, rows], sem.at[i])
                cp.start()
                copies.append(cp)
        for cp in copies:
            cp.wait()
    return _body


def kernel(x, row_weight, col_weight):
    b = x.shape[0]
    out = pl.pallas_call(
        _make_body(b),
        in_specs=[
            pl.BlockSpec(memory_space=pltpu.VMEM),
            pl.BlockSpec(memory_space=pltpu.VMEM),
        ],
        out_specs=pl.BlockSpec(memory_space=pl.ANY),
        out_shape=jax.ShapeDtypeStruct((b, _HW, _C), jnp.float32),
        scratch_shapes=[
            pltpu.VMEM((_HW, _C), jnp.float32),
            pltpu.SemaphoreType.DMA((b,)),
        ],
    )(row_weight, col_weight)
    return jnp.transpose(out.reshape(b, _H, _W, _C), (0, 3, 1, 2))


# speedup vs baseline: 6.1922x; 1.0207x over previous
import jax
import jax.numpy as jnp
from jax.experimental import pallas as pl
from jax.experimental.pallas import tpu as pltpu

_C, _H, _W = 256, 64, 64
_HALF = _C // 2
_HW = _H * _W


_NCHUNK = 8
_ROWS = _HW // _NCHUNK


def _make_body(b):
    def _body(rw_ref, cw_ref, out_ref, pos_ref, sem):
        copies = []
        for c in range(_NCHUNK):
            j = c * _ROWS + jax.lax.broadcasted_iota(jnp.int32, (_ROWS, _W), 0)
            k = jax.lax.broadcasted_iota(jnp.int32, (_ROWS, _W), 1)
            sel_w = (j % _W == k).astype(jnp.float32)
            sel_h = (j // _W == k).astype(jnp.float32)
            left = jax.lax.dot(
                sel_w, cw_ref[:_W, :], precision=jax.lax.Precision.HIGHEST)
            right = jax.lax.dot(
                sel_h, rw_ref[:_H, :], precision=jax.lax.Precision.HIGHEST)
            rows = pl.ds(c * _ROWS, _ROWS)
            pos_ref[rows, :_HALF] = left
            pos_ref[rows, _HALF:] = right
            for i in range(b):
                cp = pltpu.make_async_copy(
                    pos_ref.at[rows], out_ref.at[i, rows], sem.at[i])
                cp.start()
                copies.append(cp)
        for cp in copies:
            cp.wait()
    return _body


def kernel(x, row_weight, col_weight):
    b = x.shape[0]
    out = pl.pallas_call(
        _make_body(b),
        in_specs=[
            pl.BlockSpec(memory_space=pltpu.VMEM),
            pl.BlockSpec(memory_space=pltpu.VMEM),
        ],
        out_specs=pl.BlockSpec(memory_space=pl.ANY),
        out_shape=jax.ShapeDtypeStruct((b, _HW, _C), jnp.float32),
        scratch_shapes=[
            pltpu.VMEM((_HW, _C), jnp.float32),
            pltpu.SemaphoreType.DMA((b,)),
        ],
    )(row_weight, col_weight)
    return jnp.transpose(out.reshape(b, _H, _W, _C), (0, 3, 1, 2))


# speedup vs baseline: 6.2690x; 1.0124x over previous
import jax
import jax.numpy as jnp
from jax.experimental import pallas as pl
from jax.experimental.pallas import tpu as pltpu

_C, _H, _W = 256, 64, 64
_HALF = _C // 2
_HW = _H * _W


_NCHUNK = 16
_ROWS = _HW // _NCHUNK


def _make_body(b):
    def _body(rw_ref, cw_ref, out_ref, pos_ref, sem):
        copies = []
        for c in range(_NCHUNK):
            j = c * _ROWS + jax.lax.broadcasted_iota(jnp.int32, (_ROWS, _W), 0)
            k = jax.lax.broadcasted_iota(jnp.int32, (_ROWS, _W), 1)
            sel_w = (j % _W == k).astype(jnp.float32)
            sel_h = (j // _W == k).astype(jnp.float32)
            left = jax.lax.dot(
                sel_w, cw_ref[:_W, :], precision=jax.lax.Precision.HIGHEST)
            right = jax.lax.dot(
                sel_h, rw_ref[:_H, :], precision=jax.lax.Precision.HIGHEST)
            rows = pl.ds(c * _ROWS, _ROWS)
            pos_ref[rows, :_HALF] = left
            pos_ref[rows, _HALF:] = right
            for i in range(b):
                cp = pltpu.make_async_copy(
                    pos_ref.at[rows], out_ref.at[i, rows], sem.at[i])
                cp.start()
                copies.append(cp)
        for cp in copies:
            cp.wait()
    return _body


def kernel(x, row_weight, col_weight):
    b = x.shape[0]
    out = pl.pallas_call(
        _make_body(b),
        in_specs=[
            pl.BlockSpec(memory_space=pltpu.VMEM),
            pl.BlockSpec(memory_space=pltpu.VMEM),
        ],
        out_specs=pl.BlockSpec(memory_space=pl.ANY),
        out_shape=jax.ShapeDtypeStruct((b, _HW, _C), jnp.float32),
        scratch_shapes=[
            pltpu.VMEM((_HW, _C), jnp.float32),
            pltpu.SemaphoreType.DMA((b,)),
        ],
    )(row_weight, col_weight)
    return jnp.transpose(out.reshape(b, _H, _W, _C), (0, 3, 1, 2))


# speedup vs baseline: 6.3728x; 1.0166x over previous
import jax
import jax.numpy as jnp
from jax.experimental import pallas as pl
from jax.experimental.pallas import tpu as pltpu

_C, _H, _W = 256, 64, 64
_HALF = _C // 2
_HW = _H * _W


_NCHUNK = 32
_ROWS = _HW // _NCHUNK


def _make_body(b):
    def _body(rw_ref, cw_ref, out_ref, pos_ref, sem):
        copies = []
        for c in range(_NCHUNK):
            j = c * _ROWS + jax.lax.broadcasted_iota(jnp.int32, (_ROWS, _W), 0)
            k = jax.lax.broadcasted_iota(jnp.int32, (_ROWS, _W), 1)
            sel_w = (j % _W == k).astype(jnp.float32)
            sel_h = (j // _W == k).astype(jnp.float32)
            left = jax.lax.dot(
                sel_w, cw_ref[:_W, :], precision=jax.lax.Precision.HIGHEST)
            right = jax.lax.dot(
                sel_h, rw_ref[:_H, :], precision=jax.lax.Precision.HIGHEST)
            rows = pl.ds(c * _ROWS, _ROWS)
            pos_ref[rows, :_HALF] = left
            pos_ref[rows, _HALF:] = right
            for i in range(b):
                cp = pltpu.make_async_copy(
                    pos_ref.at[rows], out_ref.at[i, rows], sem.at[i])
                cp.start()
                copies.append(cp)
        for cp in copies:
            cp.wait()
    return _body


def kernel(x, row_weight, col_weight):
    b = x.shape[0]
    out = pl.pallas_call(
        _make_body(b),
        in_specs=[
            pl.BlockSpec(memory_space=pltpu.VMEM),
            pl.BlockSpec(memory_space=pltpu.VMEM),
        ],
        out_specs=pl.BlockSpec(memory_space=pl.ANY),
        out_shape=jax.ShapeDtypeStruct((b, _HW, _C), jnp.float32),
        scratch_shapes=[
            pltpu.VMEM((_HW, _C), jnp.float32),
            pltpu.SemaphoreType.DMA((b,)),
        ],
    )(row_weight, col_weight)
    return jnp.transpose(out.reshape(b, _H, _W, _C), (0, 3, 1, 2))
